# baseline (speedup 1.0000x reference)
import functools
import jax
import jax.numpy as jnp
from jax import lax
from jax.experimental import pallas as pl
from jax.experimental.pallas import tpu as pltpu
from jax.experimental.pallas import tpu_sc as plsc

PADDING_IDX = 1
NC = 2
NS = 16
L = 16
NW = NC * NS
CHUNK = 64


def kernel(x, weights):
    bsz, seq_len = x.shape
    embed_dim = weights.shape[1]
    n_rows = bsz * seq_len
    rows_per_w = n_rows // NW
    n_chunks = rows_per_w // CHUNK

    x_flat = x.reshape(n_rows)
    mesh = plsc.VectorSubcoreMesh(core_axis_name="c", subcore_axis_name="s")

    @functools.partial(
        pl.kernel,
        mesh=mesh,
        out_type=jax.ShapeDtypeStruct((n_rows, embed_dim), jnp.float32),
        scratch_types=[
            pltpu.VMEM((rows_per_w,), jnp.int32),
            pltpu.VMEM((CHUNK,), jnp.int32),
            pltpu.VMEM((CHUNK, embed_dim), jnp.float32),
            pltpu.SemaphoreType.DMA,
        ],
    )
    def sc_gather(x_hbm, w_hbm, out_hbm, x_v, idx_v, rows_v, sem):
        wid = lax.axis_index("s") * NC + lax.axis_index("c")
        base = wid * rows_per_w
        base_s = lax.rem(base, seq_len)
        pltpu.sync_copy(x_hbm.at[pl.ds(base, rows_per_w)], x_v)
        iota = lax.iota(jnp.int32, L)

        def chunk_body(c, _):
            cbase = c * CHUNK

            def idx_body(j, _):
                o = cbase + j * L
                xv = x_v[pl.ds(o, L)]
                pos = base_s + o + (PADDING_IDX + 1) + iota
                idx_v[pl.ds(j * L, L)] = jnp.where(
                    xv != PADDING_IDX, pos, PADDING_IDX)
                return 0

            lax.fori_loop(0, CHUNK // L, idx_body, 0)
            pltpu.async_copy(w_hbm.at[idx_v], rows_v, sem).wait()
            pltpu.sync_copy(rows_v, out_hbm.at[pl.ds(base + cbase, CHUNK)])
            return 0

        lax.fori_loop(0, n_chunks, chunk_body, 0)

    out = sc_gather(x_flat, weights)
    return out.reshape(bsz, seq_len, embed_dim)


# speedup vs baseline: 1.1049x; 1.1049x over previous
import functools
import jax
import jax.numpy as jnp
from jax import lax
from jax.experimental import pallas as pl
from jax.experimental.pallas import tpu as pltpu
from jax.experimental.pallas import tpu_sc as plsc

PADDING_IDX = 1
NC = 2
NS = 16
L = 16
NW = NC * NS
CHUNK = 32


def kernel(x, weights):
    bsz, seq_len = x.shape
    embed_dim = weights.shape[1]
    n_rows = bsz * seq_len
    rows_per_w = n_rows // NW
    n_chunks = rows_per_w // CHUNK

    x_flat = x.reshape(n_rows)
    mesh = plsc.VectorSubcoreMesh(core_axis_name="c", subcore_axis_name="s")

    @functools.partial(
        pl.kernel,
        mesh=mesh,
        out_type=jax.ShapeDtypeStruct((n_rows, embed_dim), jnp.float32),
        scratch_types=[
            pltpu.VMEM((rows_per_w,), jnp.int32),
            pltpu.VMEM((CHUNK,), jnp.int32),
            pltpu.VMEM((CHUNK,), jnp.int32),
            pltpu.VMEM((CHUNK, embed_dim), jnp.float32),
            pltpu.VMEM((CHUNK, embed_dim), jnp.float32),
            pltpu.SemaphoreType.DMA,
            pltpu.SemaphoreType.DMA,
            pltpu.SemaphoreType.DMA,
            pltpu.SemaphoreType.DMA,
        ],
    )
    def sc_gather(x_hbm, w_hbm, out_hbm, x_v, idx0, idx1, rows0, rows1,
                  gsem0, gsem1, ssem0, ssem1):
        idx_v = (idx0, idx1)
        rows_v = (rows0, rows1)
        gsem = (gsem0, gsem1)
        ssem = (ssem0, ssem1)

        wid = lax.axis_index("s") * NC + lax.axis_index("c")
        base = wid * rows_per_w
        base_s = lax.rem(base, seq_len)
        pltpu.sync_copy(x_hbm.at[pl.ds(base, rows_per_w)], x_v)
        iota = lax.iota(jnp.int32, L)

        def compute_idx(c, buf):
            def idx_body(j, _):
                o = c * CHUNK + j * L
                xv = x_v[pl.ds(o, L)]
                pos = base_s + o + (PADDING_IDX + 1) + iota
                idx_v[buf][pl.ds(j * L, L)] = jnp.where(
                    xv != PADDING_IDX, pos, PADDING_IDX)
                return 0
            lax.fori_loop(0, CHUNK // L, idx_body, 0)

        def start_gather(buf):
            pltpu.async_copy(w_hbm.at[idx_v[buf]], rows_v[buf], gsem[buf])

        def wait_gather(buf):
            pltpu.make_async_copy(
                w_hbm.at[idx_v[buf]], rows_v[buf], gsem[buf]).wait()

        def start_scatter(c, buf):
            pltpu.async_copy(
                rows_v[buf], out_hbm.at[pl.ds(base + c * CHUNK, CHUNK)],
                ssem[buf])

        def wait_scatter(c, buf):
            pltpu.make_async_copy(
                rows_v[buf], out_hbm.at[pl.ds(base + c * CHUNK, CHUNK)],
                ssem[buf]).wait()

        compute_idx(0, 0)
        start_gather(0)

        def pair_body(g, _):
            for b in range(2):
                c = g * 2 + b
                nb = 1 - b
                @pl.when(c >= 1)
                def _():
                    wait_scatter(c - 1, nb)

                @pl.when(c + 1 < n_chunks)
                def _():
                    compute_idx(c + 1, nb)
                    start_gather(nb)

                wait_gather(b)
                start_scatter(c, b)
            return 0

        lax.fori_loop(0, n_chunks // 2, pair_body, 0)
        wait_scatter(n_chunks - 1, (n_chunks - 1) % 2)

    out = sc_gather(x_flat, weights)
    return out.reshape(bsz, seq_len, embed_dim)
